# baseline (speedup 1.0000x reference)
import functools
import math

import jax
import jax.numpy as jnp
from jax import lax
from jax.experimental import pallas as pl
from jax.experimental.pallas import tpu as pltpu
from jax.experimental.pallas import tpu_sc as plsc

_T, _B, _C, _W = 2048, 8, 512, 64
_PE_ROWS = 32
_NC, _NS = 2, 16
_NW = _NC * _NS
_TSC = 512
_ROWS_SC = _TSC * _B
_RPW = _ROWS_SC // _NW
_CHUNK = 128
_NCHUNK = _RPW // _CHUNK
_NBUF = min(3, _NCHUNK)
_TBLK = 256


def _pe_tab():
    pos = jnp.arange(_PE_ROWS, dtype=jnp.float32)[:, None]
    div = jnp.exp(
        jnp.arange(0, _C, 2, dtype=jnp.float32) * (-math.log(10000.0) / _C)
    )
    ang = pos * div
    pe = jnp.stack([jnp.sin(ang), jnp.cos(ang)], axis=-1).reshape(_PE_ROWS, _C)
    pe = pe.at[_PE_ROWS - 1].set(0.0)
    return pe


def _idx_body(dur_ref, o_ref):
    i = pl.program_id(0)
    dur = dur_ref[...].astype(jnp.float32)
    tri = (
        jax.lax.broadcasted_iota(jnp.int32, (_W, _W), 0)
        <= jax.lax.broadcasted_iota(jnp.int32, (_W, _W), 1)
    ).astype(jnp.float32)
    csum = jnp.dot(dur, tri, preferred_element_type=jnp.float32).astype(
        jnp.int32
    )

    t3 = jax.lax.broadcasted_iota(jnp.int32, (_TBLK, _B, _W), 0) + i * _TBLK
    le = csum[None, :, :] <= t3
    start = jnp.max(jnp.where(le, csum[None, :, :], 0), axis=2)

    t2 = jax.lax.broadcasted_iota(jnp.int32, (_TBLK, _B), 0) + i * _TBLK
    total = csum[:, _W - 1]
    mask = t2 < total[None, :]
    win = jnp.where(mask, t2 - start, _PE_ROWS - 1)
    o_ref[...] = jnp.broadcast_to(win[:, :, None], (_TBLK, _B, 16))


def _token_pe_idx(text_duration):
    return pl.pallas_call(
        _idx_body,
        grid=(_TSC // _TBLK,),
        in_specs=[pl.BlockSpec((_B, _W), lambda i: (0, 0))],
        out_specs=pl.BlockSpec((_TBLK, _B, 16), lambda i: (i, 0, 0)),
        out_shape=jax.ShapeDtypeStruct((_TSC, _B, 16), jnp.int32),
    )(text_duration)


def _tc_add_body(dur_ref, x_ref, pe_ref, o_ref):
    i = pl.program_id(0)
    dur = dur_ref[...].astype(jnp.float32)
    tri = (
        jax.lax.broadcasted_iota(jnp.int32, (_W, _W), 0)
        <= jax.lax.broadcasted_iota(jnp.int32, (_W, _W), 1)
    ).astype(jnp.float32)
    csum = jnp.dot(dur, tri, preferred_element_type=jnp.float32).astype(
        jnp.int32
    )
    t0 = _TSC + i * _TBLK
    t3 = jax.lax.broadcasted_iota(jnp.int32, (_TBLK, _B, _W), 0) + t0
    le = csum[None, :, :] <= t3
    start = jnp.max(jnp.where(le, csum[None, :, :], 0), axis=2)
    t2 = jax.lax.broadcasted_iota(jnp.int32, (_TBLK, _B), 0) + t0
    total = csum[:, _W - 1]
    mask = t2 < total[None, :]
    win = jnp.where(mask, t2 - start, _PE_ROWS - 1)
    oh = (
        win[:, :, None]
        == jax.lax.broadcasted_iota(jnp.int32, (_TBLK, _B, _PE_ROWS), 2)
    )
    ohf = oh.astype(jnp.float32).reshape(_TBLK * _B, _PE_ROWS)
    add = jnp.dot(ohf, pe_ref[...], preferred_element_type=jnp.float32)
    o_ref[...] = x_ref[...] + add.reshape(_TBLK, _B, _C)


def _tc_add_body_alias(dur_ref, x_ref, pe_ref, scout_ref, o_ref):
    del scout_ref
    _tc_add_body(dur_ref, x_ref, pe_ref, o_ref)


def _tc_add(x, text_duration, pe, scout):
    nblk = (_T - _TSC) // _TBLK
    off = _TSC // _TBLK
    return pl.pallas_call(
        _tc_add_body_alias,
        grid=(nblk,),
        in_specs=[
            pl.BlockSpec((_B, _W), lambda i: (0, 0)),
            pl.BlockSpec((_TBLK, _B, _C), lambda i: (i + off, 0, 0)),
            pl.BlockSpec((_PE_ROWS, _C), lambda i: (0, 0)),
            pl.BlockSpec(memory_space=pl.ANY),
        ],
        out_specs=pl.BlockSpec((_TBLK, _B, _C), lambda i: (i + off, 0, 0)),
        out_shape=jax.ShapeDtypeStruct((_T, _B, _C), jnp.float32),
        input_output_aliases={3: 0},
    )(text_duration, x, pe, scout)


def _sc_body(x_hbm, idx_hbm, pe_hbm, out_hbm, idx_v, pe_v, *bufs_sems):
    wid = lax.axis_index("s") * _NC + lax.axis_index("c")
    row0 = wid * _RPW
    pltpu.sync_copy(idx_hbm.at[wid], idx_v)
    pltpu.sync_copy(pe_hbm, pe_v)

    bxs = bufs_sems[:_NBUF]
    semis = bufs_sems[_NBUF : 2 * _NBUF]
    semos = bufs_sems[2 * _NBUF : 3 * _NBUF]

    def stream_in(ch):
        s = ch % _NBUF
        return pltpu.async_copy(
            x_hbm.at[pl.ds(row0 + ch * _CHUNK, _CHUNK)], bxs[s], semis[s]
        )

    ins = [None] * _NCHUNK
    outs = [None] * _NCHUNK
    ins[0] = stream_in(0)
    if _NCHUNK > 1:
        ins[1] = stream_in(1)
    for ch in range(_NCHUNK):
        s = ch % _NBUF
        bx = bxs[s]
        ins[ch].wait()

        def row_add(r, carry):
            rr = ch * _CHUNK + r
            wv = idx_v[rr >> 3, pl.ds((rr & 7) * 16, 16)]
            w = jnp.squeeze(lax.slice(wv, (0,), (1,)))
            for j in range(_C // 16):
                plsc.addupdate(
                    bx.at[r, pl.ds(j * 16, 16)], pe_v[w, pl.ds(j * 16, 16)]
                )
            return carry

        lax.fori_loop(0, _CHUNK, row_add, 0)
        outs[ch] = pltpu.async_copy(
            bx, out_hbm.at[pl.ds(row0 + ch * _CHUNK, _CHUNK)], semos[s]
        )
        if ch + 2 < _NCHUNK:
            if ch >= 1:
                outs[ch - 1].wait()
            ins[ch + 2] = stream_in(ch + 2)
    for k in range(max(0, _NCHUNK - 3), _NCHUNK):
        outs[k].wait()


def kernel(x, text_duration, train):
    del train
    idx = _token_pe_idx(text_duration).reshape(_NW, _RPW // 8, 128)
    xr = x.reshape(_T * _B, _C)
    pe = _pe_tab()
    mesh = plsc.VectorSubcoreMesh(core_axis_name="c", subcore_axis_name="s")
    run = functools.partial(
        pl.kernel,
        mesh=mesh,
        out_type=jax.ShapeDtypeStruct((_T * _B, _C), jnp.float32),
        scratch_types=(
            [
                pltpu.VMEM((_RPW // 8, 128), jnp.int32),
                pltpu.VMEM((_PE_ROWS, _C), jnp.float32),
            ]
            + [pltpu.VMEM((_CHUNK, _C), jnp.float32)] * _NBUF
            + [pltpu.SemaphoreType.DMA] * (2 * _NBUF)
        ),
    )(_sc_body)
    out_sc = run(xr, idx, pe)
    return _tc_add(x, text_duration, pe, out_sc.reshape(_T, _B, _C))


# speedup vs baseline: 1.1477x; 1.1477x over previous
import functools
import math

import jax
import jax.numpy as jnp
from jax import lax
from jax.experimental import pallas as pl
from jax.experimental.pallas import tpu as pltpu
from jax.experimental.pallas import tpu_sc as plsc

_T, _B, _C, _W = 2048, 8, 512, 64
_PE_ROWS = 32
_NC, _NS = 2, 16
_NW = _NC * _NS
_TSC = 512
_ROWS_SC = _TSC * _B
_RPW = _ROWS_SC // _NW
_CHUNK = 64
_NCHUNK = _RPW // _CHUNK
_NBUF = min(3, _NCHUNK)
_TBLK = 256


def _pe_tab():
    pos = jnp.arange(_PE_ROWS, dtype=jnp.float32)[:, None]
    div = jnp.exp(
        jnp.arange(0, _C, 2, dtype=jnp.float32) * (-math.log(10000.0) / _C)
    )
    ang = pos * div
    pe = jnp.stack([jnp.sin(ang), jnp.cos(ang)], axis=-1).reshape(_PE_ROWS, _C)
    pe = pe.at[_PE_ROWS - 1].set(0.0)
    return pe


def _idx_body(dur_ref, o_ref):
    i = pl.program_id(0)
    dur = dur_ref[...].astype(jnp.float32)
    tri = (
        jax.lax.broadcasted_iota(jnp.int32, (_W, _W), 0)
        <= jax.lax.broadcasted_iota(jnp.int32, (_W, _W), 1)
    ).astype(jnp.float32)
    csum = jnp.dot(dur, tri, preferred_element_type=jnp.float32).astype(
        jnp.int32
    )

    t3 = jax.lax.broadcasted_iota(jnp.int32, (_TBLK, _B, _W), 0) + i * _TBLK
    le = csum[None, :, :] <= t3
    start = jnp.max(jnp.where(le, csum[None, :, :], 0), axis=2)

    t2 = jax.lax.broadcasted_iota(jnp.int32, (_TBLK, _B), 0) + i * _TBLK
    total = csum[:, _W - 1]
    mask = t2 < total[None, :]
    win = jnp.where(mask, t2 - start, _PE_ROWS - 1)
    o_ref[...] = jnp.broadcast_to(win[:, :, None], (_TBLK, _B, 16))


def _token_pe_idx(text_duration):
    return pl.pallas_call(
        _idx_body,
        grid=(_TSC // _TBLK,),
        in_specs=[pl.BlockSpec((_B, _W), lambda i: (0, 0))],
        out_specs=pl.BlockSpec((_TBLK, _B, 16), lambda i: (i, 0, 0)),
        out_shape=jax.ShapeDtypeStruct((_TSC, _B, 16), jnp.int32),
    )(text_duration)


def _tc_add_body(dur_ref, x_ref, pe_ref, o_ref):
    i = pl.program_id(0)
    dur = dur_ref[...].astype(jnp.float32)
    tri = (
        jax.lax.broadcasted_iota(jnp.int32, (_W, _W), 0)
        <= jax.lax.broadcasted_iota(jnp.int32, (_W, _W), 1)
    ).astype(jnp.float32)
    csum = jnp.dot(dur, tri, preferred_element_type=jnp.float32).astype(
        jnp.int32
    )
    t0 = _TSC + i * _TBLK
    t3 = jax.lax.broadcasted_iota(jnp.int32, (_TBLK, _B, _W), 0) + t0
    le = csum[None, :, :] <= t3
    start = jnp.max(jnp.where(le, csum[None, :, :], 0), axis=2)
    t2 = jax.lax.broadcasted_iota(jnp.int32, (_TBLK, _B), 0) + t0
    total = csum[:, _W - 1]
    mask = t2 < total[None, :]
    win = jnp.where(mask, t2 - start, _PE_ROWS - 1)
    oh = (
        win[:, :, None]
        == jax.lax.broadcasted_iota(jnp.int32, (_TBLK, _B, _PE_ROWS), 2)
    )
    ohf = oh.astype(jnp.float32).reshape(_TBLK * _B, _PE_ROWS)
    add = jnp.dot(ohf, pe_ref[...], preferred_element_type=jnp.float32)
    o_ref[...] = x_ref[...] + add.reshape(_TBLK, _B, _C)


def _tc_add_body_alias(dur_ref, x_ref, pe_ref, scout_ref, o_ref):
    del scout_ref
    _tc_add_body(dur_ref, x_ref, pe_ref, o_ref)


def _tc_add(x, text_duration, pe, scout):
    nblk = (_T - _TSC) // _TBLK
    off = _TSC // _TBLK
    return pl.pallas_call(
        _tc_add_body_alias,
        grid=(nblk,),
        in_specs=[
            pl.BlockSpec((_B, _W), lambda i: (0, 0)),
            pl.BlockSpec((_TBLK, _B, _C), lambda i: (i + off, 0, 0)),
            pl.BlockSpec((_PE_ROWS, _C), lambda i: (0, 0)),
            pl.BlockSpec(memory_space=pl.ANY),
        ],
        out_specs=pl.BlockSpec((_TBLK, _B, _C), lambda i: (i + off, 0, 0)),
        out_shape=jax.ShapeDtypeStruct((_T, _B, _C), jnp.float32),
        input_output_aliases={3: 0},
    )(text_duration, x, pe, scout)


def _sc_body(x_hbm, idx_hbm, pe_hbm, out_hbm, idx_v, pe_v, *bufs_sems):
    wid = lax.axis_index("s") * _NC + lax.axis_index("c")
    row0 = wid * _RPW
    pltpu.sync_copy(idx_hbm.at[wid], idx_v)
    pltpu.sync_copy(pe_hbm, pe_v)

    bxs = bufs_sems[:_NBUF]
    semis = bufs_sems[_NBUF : 2 * _NBUF]
    semos = bufs_sems[2 * _NBUF : 3 * _NBUF]

    def stream_in(ch):
        s = ch % _NBUF
        return pltpu.async_copy(
            x_hbm.at[pl.ds(row0 + ch * _CHUNK, _CHUNK)], bxs[s], semis[s]
        )

    ins = [None] * _NCHUNK
    outs = [None] * _NCHUNK
    ins[0] = stream_in(0)
    if _NCHUNK > 1:
        ins[1] = stream_in(1)
    for ch in range(_NCHUNK):
        s = ch % _NBUF
        bx = bxs[s]
        ins[ch].wait()

        @plsc.parallel_loop(0, _CHUNK, 1, unroll=4)
        def row_add(r):
            rr = ch * _CHUNK + r
            wv = idx_v[rr >> 3, pl.ds((rr & 7) * 16, 16)]
            w = jnp.squeeze(lax.slice(wv, (0,), (1,)))
            for j in range(_C // 16):
                plsc.addupdate(
                    bx.at[r, pl.ds(j * 16, 16)], pe_v[w, pl.ds(j * 16, 16)]
                )
        outs[ch] = pltpu.async_copy(
            bx, out_hbm.at[pl.ds(row0 + ch * _CHUNK, _CHUNK)], semos[s]
        )
        if ch + 2 < _NCHUNK:
            if ch >= 1:
                outs[ch - 1].wait()
            ins[ch + 2] = stream_in(ch + 2)
    for k in range(max(0, _NCHUNK - 3), _NCHUNK):
        outs[k].wait()


def kernel(x, text_duration, train):
    del train
    idx = _token_pe_idx(text_duration).reshape(_NW, _RPW // 8, 128)
    xr = x.reshape(_T * _B, _C)
    pe = _pe_tab()
    mesh = plsc.VectorSubcoreMesh(core_axis_name="c", subcore_axis_name="s")
    run = functools.partial(
        pl.kernel,
        mesh=mesh,
        out_type=jax.ShapeDtypeStruct((_T * _B, _C), jnp.float32),
        scratch_types=(
            [
                pltpu.VMEM((_RPW // 8, 128), jnp.int32),
                pltpu.VMEM((_PE_ROWS, _C), jnp.float32),
            ]
            + [pltpu.VMEM((_CHUNK, _C), jnp.float32)] * _NBUF
            + [pltpu.SemaphoreType.DMA] * (2 * _NBUF)
        ),
    )(_sc_body)
    out_sc = run(xr, idx, pe)
    return _tc_add(x, text_duration, pe, out_sc.reshape(_T, _B, _C))
